# speedup vs baseline: 9.3176x; 9.3176x over previous
import jax
import jax.numpy as jnp
from jax import lax
from jax.experimental import pallas as pl
from jax.experimental.pallas import tpu as pltpu

_R = 70
_H = 512
_S = 128
_STRIDE = 3
_CHUNK = 80


def _propose_kernel(score_ref, fake_ref, reala_ref,
                    fbr_ref, rar_ref, fbf_ref, raf_ref, fabf_ref, rabr_ref):
    s = score_ref[0, 0]
    ri = lax.broadcasted_iota(jnp.int32, (_S, _S), 0)
    ci = lax.broadcasted_iota(jnp.int32, (_S, _S), 1)
    flat = ri * _S + ci
    big = jnp.int32(1 << 30)
    vmax = jnp.max(s)
    vmin = jnp.min(s)
    imax = jnp.min(jnp.where(s == vmax, flat, big))
    imin = jnp.min(jnp.where(s == vmin, flat, big))
    rr = jnp.where(vmax > 0.0, imax // _S, 0) * _STRIDE + _R
    cr = jnp.where(vmax > 0.0, imax % _S, 0) * _STRIDE + _R
    rf = jnp.where(vmin < 1.0, imin // _S, 1) * _STRIDE + _R
    cf = jnp.where(vmin < 1.0, imin % _S, 1) * _STRIDE + _R

    def crop_pair(r0, c0):
        ra = jnp.minimum((r0 >> 3) << 3, _H - _CHUNK)
        jc = lax.broadcasted_iota(jnp.int32, (_H, _R), 0)
        kc = lax.broadcasted_iota(jnp.int32, (_H, _R), 1)
        csel = (jc == c0 + kc).astype(jnp.float32)
        ir = lax.broadcasted_iota(jnp.int32, (_R, _CHUNK), 0)
        jr = lax.broadcasted_iota(jnp.int32, (_R, _CHUNK), 1)
        rsel = ((ra + jr == r0 + ir) & (r0 + ir <= _H - 1)).astype(jnp.float32)

        def one(img_ref):
            outs = []
            for c in range(3):
                rows = img_ref[0, c, pl.ds(pl.multiple_of(ra, 8), _CHUNK), :]
                t = jnp.dot(rows, csel, preferred_element_type=jnp.float32)
                outs.append(jnp.dot(rsel, t, preferred_element_type=jnp.float32))
            return outs

        return one(fake_ref), one(reala_ref)

    fbr, rar = crop_pair(rr, cr)
    fbf, raf = crop_pair(rf, cf)
    for c in range(3):
        fbr_ref[0, c] = fbr[c]
        rar_ref[0, c] = rar[c]
        fbf_ref[0, c] = fbf[c]
        raf_ref[0, c] = raf[c]
        fabf_ref[0, c] = raf[c]
        fabf_ref[0, 3 + c] = fbf[c]
        rabr_ref[0, c] = rar[c]
        rabr_ref[0, 3 + c] = fbr[c]


def kernel(real_B, fake_B, real_A, score_map):
    del real_B
    B = score_map.shape[0]
    f32 = jnp.float32
    crop3 = jax.ShapeDtypeStruct((B, 3, _R, _R), f32)
    crop6 = jax.ShapeDtypeStruct((B, 6, _R, _R), f32)
    spec3 = pl.BlockSpec((1, 3, _R, _R), lambda b: (b, 0, 0, 0))
    spec6 = pl.BlockSpec((1, 6, _R, _R), lambda b: (b, 0, 0, 0))
    outs = pl.pallas_call(
        _propose_kernel,
        out_shape=(crop3, crop3, crop3, crop3, crop6, crop6),
        grid=(B,),
        in_specs=[
            pl.BlockSpec((1, 1, _S, _S), lambda b: (b, 0, 0, 0)),
            pl.BlockSpec((1, 3, _H, _H), lambda b: (b, 0, 0, 0)),
            pl.BlockSpec((1, 3, _H, _H), lambda b: (b, 0, 0, 0)),
        ],
        out_specs=(spec3, spec3, spec3, spec3, spec6, spec6),
        compiler_params=pltpu.CompilerParams(
            dimension_semantics=("parallel",),
        ),
        name="drpan_proposal",
    )(score_map, fake_B, real_A)
    return tuple(outs)
